# baseline (speedup 1.0000x reference)
import functools

import jax
import jax.numpy as jnp
from jax import lax
from jax.experimental import pallas as pl
from jax.experimental.pallas import tpu as pltpu
from jax.experimental.pallas import tpu_sc as plsc

N_NODES = 10000
N_EDGES = 160000
D_FEAT = 256

_NC, _NS, _L = 2, 16, 16
_NW = _NC * _NS
_CT = 128
_MAIN_E = 39 * _CT
_MAIN_CHUNKS = _MAIN_E // _L
_TAIL_E0 = _NW * _MAIN_E
_TAIL_CHUNKS = _CT // _L
_BUF_E = _MAIN_E + _CT

_NB = 1024


def _node_proj_body(h_ref, w_ref, b_ref, s1_ref, s2_ref):
    hb = h_ref[...]
    half_b = 0.5 * b_ref[0]
    w1 = w_ref[0:D_FEAT].reshape(1, D_FEAT)
    w2 = w_ref[D_FEAT:2 * D_FEAT].reshape(1, D_FEAT)
    dn = (((1,), (1,)), ((), ()))
    r1 = lax.dot_general(w1, hb, dn, preferred_element_type=jnp.float32)
    r2 = lax.dot_general(w2, hb, dn, preferred_element_type=jnp.float32)
    s1_ref[...] = r1.reshape(_NB) + half_b
    s2_ref[...] = r2.reshape(_NB) + half_b


def _node_projections(h, W, b):
    return pl.pallas_call(
        _node_proj_body,
        grid=(pl.cdiv(N_NODES, _NB),),
        in_specs=[
            pl.BlockSpec((_NB, D_FEAT), lambda i: (i, 0)),
            pl.BlockSpec((2 * D_FEAT,), lambda i: (0,)),
            pl.BlockSpec(memory_space=pltpu.SMEM),
        ],
        out_specs=[
            pl.BlockSpec((_NB,), lambda i: (i,)),
            pl.BlockSpec((_NB,), lambda i: (i,)),
        ],
        out_shape=[
            jax.ShapeDtypeStruct((N_NODES,), jnp.float32),
            jax.ShapeDtypeStruct((N_NODES,), jnp.float32),
        ],
    )(h, W.reshape(2 * D_FEAT), b)


_sc_mesh = plsc.VectorSubcoreMesh(core_axis_name="c", subcore_axis_name="s")


@functools.partial(
    pl.kernel,
    out_type=jax.ShapeDtypeStruct((N_EDGES,), jnp.float32),
    mesh=_sc_mesh,
    compiler_params=pltpu.CompilerParams(needs_layout_passes=False),
    scratch_types=[
        pltpu.VMEM((N_NODES,), jnp.float32),
        pltpu.VMEM((N_NODES,), jnp.float32),
        pltpu.VMEM((2, _BUF_E), jnp.int32),
        pltpu.VMEM((_BUF_E,), jnp.float32),
        pltpu.SemaphoreType.DMA,
    ],
)
def _edge_score_kernel(s1_hbm, s2_hbm, ei_hbm, out_hbm,
                       s1_v, s2_v, ei_v, o_v, sem):
    wid = lax.axis_index("s") * _NC + lax.axis_index("c")
    base = wid * _MAIN_E
    has_tail = wid < 2

    d1 = pltpu.async_copy(s1_hbm, s1_v, sem)
    d2 = pltpu.async_copy(s2_hbm, s2_v, sem)
    d3 = pltpu.async_copy(ei_hbm.at[:, pl.ds(base, _MAIN_E)],
                          ei_v.at[:, pl.ds(0, _MAIN_E)], sem)

    @pl.when(has_tail)
    def _tail_idx():
        pltpu.async_copy(ei_hbm.at[:, pl.ds(_TAIL_E0 + wid * _CT, _CT)],
                         ei_v.at[:, pl.ds(_MAIN_E, _CT)], sem).wait()

    d1.wait()
    d2.wait()
    d3.wait()

    def score_chunk(off):
        si = ei_v[0, pl.ds(off, _L)]
        di = ei_v[1, pl.ds(off, _L)]
        x = plsc.load_gather(s1_v, [si]) + plsc.load_gather(s2_v, [di])
        o_v[pl.ds(off, _L)] = 1.0 / (1.0 + jnp.exp(-x))

    @plsc.parallel_loop(0, _MAIN_CHUNKS, unroll=8)
    def _main(i):
        score_chunk(pl.multiple_of(i * _L, _L))

    @pl.when(has_tail)
    def _tail():
        @plsc.parallel_loop(0, _TAIL_CHUNKS, unroll=8)
        def _t(i):
            score_chunk(pl.multiple_of(_MAIN_E + i * _L, _L))

    pltpu.sync_copy(o_v.at[pl.ds(0, _MAIN_E)],
                    out_hbm.at[pl.ds(base, _MAIN_E)])

    @pl.when(has_tail)
    def _tail_out():
        pltpu.sync_copy(o_v.at[pl.ds(_MAIN_E, _CT)],
                        out_hbm.at[pl.ds(_TAIL_E0 + wid * _CT, _CT)])


def kernel(h, edge_index, W, b):
    s1, s2 = _node_projections(h, W, b)
    return _edge_score_kernel(s1, s2, edge_index)


# speedup vs baseline: 1.0193x; 1.0193x over previous
import functools

import jax
import jax.numpy as jnp
from jax import lax
from jax.experimental import pallas as pl
from jax.experimental.pallas import tpu as pltpu
from jax.experimental.pallas import tpu_sc as plsc

N_NODES = 10000
N_EDGES = 160000
D_FEAT = 256

_NC, _NS, _L = 2, 16, 16
_NW = _NC * _NS
_CT = 128
_MAIN_E = 39 * _CT
_MAIN_CHUNKS = _MAIN_E // _L
_TAIL_E0 = _NW * _MAIN_E
_TAIL_CHUNKS = _CT // _L
_BUF_E = _MAIN_E + _CT

_NB = 1024
_N_CHUNKS = (N_NODES + _NB - 1) // _NB
_LAST = N_NODES - (_N_CHUNKS - 1) * _NB


def _node_proj_body(h_hbm, w_ref, b_ref, s1_ref, s2_ref, buf, sems):
    half_b = 0.5 * b_ref[0]
    w1 = w_ref[0:D_FEAT].reshape(1, D_FEAT)
    w2 = w_ref[D_FEAT:2 * D_FEAT].reshape(1, D_FEAT)
    dn = (((1,), (1,)), ((), ()))

    def copy(i):
        sz = _NB if i < _N_CHUNKS - 1 else _LAST
        return pltpu.make_async_copy(
            h_hbm.at[pl.ds(i * _NB, sz), :],
            buf.at[i % 2, pl.ds(0, sz), :],
            sems.at[i % 2])

    copy(0).start()
    for i in range(_N_CHUNKS):
        if i + 1 < _N_CHUNKS:
            copy(i + 1).start()
        copy(i).wait()
        sz = _NB if i < _N_CHUNKS - 1 else _LAST
        hb = buf[i % 2, 0:sz, :]
        r1 = lax.dot_general(w1, hb, dn, preferred_element_type=jnp.float32)
        r2 = lax.dot_general(w2, hb, dn, preferred_element_type=jnp.float32)
        s1_ref[pl.ds(i * _NB, sz)] = r1.reshape(sz) + half_b
        s2_ref[pl.ds(i * _NB, sz)] = r2.reshape(sz) + half_b


def _node_projections(h, W, b):
    return pl.pallas_call(
        _node_proj_body,
        in_specs=[
            pl.BlockSpec(memory_space=pl.ANY),
            pl.BlockSpec(memory_space=pltpu.VMEM),
            pl.BlockSpec(memory_space=pltpu.SMEM),
        ],
        out_shape=[
            jax.ShapeDtypeStruct((N_NODES,), jnp.float32),
            jax.ShapeDtypeStruct((N_NODES,), jnp.float32),
        ],
        scratch_shapes=[
            pltpu.VMEM((2, _NB, D_FEAT), jnp.float32),
            pltpu.SemaphoreType.DMA((2,)),
        ],
    )(pltpu.with_memory_space_constraint(h, pltpu.MemorySpace.HBM),
      W.reshape(2 * D_FEAT), b)


_sc_mesh = plsc.VectorSubcoreMesh(core_axis_name="c", subcore_axis_name="s")


@functools.partial(
    pl.kernel,
    out_type=jax.ShapeDtypeStruct((N_EDGES,), jnp.float32),
    mesh=_sc_mesh,
    compiler_params=pltpu.CompilerParams(needs_layout_passes=False),
    scratch_types=[
        pltpu.VMEM((N_NODES,), jnp.float32),
        pltpu.VMEM((N_NODES,), jnp.float32),
        pltpu.VMEM((2, _BUF_E), jnp.int32),
        pltpu.VMEM((_BUF_E,), jnp.float32),
        pltpu.SemaphoreType.DMA,
    ],
)
def _edge_score_kernel(s1_hbm, s2_hbm, ei_hbm, out_hbm,
                       s1_v, s2_v, ei_v, o_v, sem):
    wid = lax.axis_index("s") * _NC + lax.axis_index("c")
    base = wid * _MAIN_E
    has_tail = wid < 2

    d1 = pltpu.async_copy(s1_hbm, s1_v, sem)
    d2 = pltpu.async_copy(s2_hbm, s2_v, sem)
    d3 = pltpu.async_copy(ei_hbm.at[:, pl.ds(base, _MAIN_E)],
                          ei_v.at[:, pl.ds(0, _MAIN_E)], sem)

    @pl.when(has_tail)
    def _tail_idx():
        pltpu.async_copy(ei_hbm.at[:, pl.ds(_TAIL_E0 + wid * _CT, _CT)],
                         ei_v.at[:, pl.ds(_MAIN_E, _CT)], sem).wait()

    d1.wait()
    d2.wait()
    d3.wait()

    def score_chunk(off):
        si = ei_v[0, pl.ds(off, _L)]
        di = ei_v[1, pl.ds(off, _L)]
        x = plsc.load_gather(s1_v, [si]) + plsc.load_gather(s2_v, [di])
        o_v[pl.ds(off, _L)] = 1.0 / (1.0 + jnp.exp(-x))

    @plsc.parallel_loop(0, _MAIN_CHUNKS, unroll=8)
    def _main(i):
        score_chunk(pl.multiple_of(i * _L, _L))

    @pl.when(has_tail)
    def _tail():
        @plsc.parallel_loop(0, _TAIL_CHUNKS, unroll=8)
        def _t(i):
            score_chunk(pl.multiple_of(_MAIN_E + i * _L, _L))

    pltpu.sync_copy(o_v.at[pl.ds(0, _MAIN_E)],
                    out_hbm.at[pl.ds(base, _MAIN_E)])

    @pl.when(has_tail)
    def _tail_out():
        pltpu.sync_copy(o_v.at[pl.ds(_MAIN_E, _CT)],
                        out_hbm.at[pl.ds(_TAIL_E0 + wid * _CT, _CT)])


def kernel(h, edge_index, W, b):
    s1, s2 = _node_projections(h, W, b)
    return _edge_score_kernel(s1, s2, edge_index)


# speedup vs baseline: 1.1228x; 1.1016x over previous
import functools

import jax
import jax.numpy as jnp
from jax import lax
from jax.experimental import pallas as pl
from jax.experimental.pallas import tpu as pltpu
from jax.experimental.pallas import tpu_sc as plsc

N_NODES = 10000
N_EDGES = 160000
D_FEAT = 256

_NC, _NS, _L = 2, 16, 16
_NW = _NC * _NS
_CT = 128
_MAIN_E = 39 * _CT
_MAIN_CHUNKS = _MAIN_E // _L
_TAIL_E0 = _NW * _MAIN_E
_TAIL_CHUNKS = _CT // _L
_BUF_E = _MAIN_E + _CT

def _node_proj_body(h_ref, w_ref, b_ref, s1_ref, s2_ref):
    w1 = w_ref[0:D_FEAT].reshape(1, D_FEAT)
    w2 = w_ref[D_FEAT:2 * D_FEAT].reshape(1, D_FEAT)
    wc = jnp.concatenate([w1, w2], axis=0)
    dn = (((1,), (1,)), ((), ()))
    r = lax.dot_general(wc, h_ref[...], dn, preferred_element_type=jnp.float32)
    half_b = 0.5 * b_ref[0]
    s1_ref[...] = r[0:1, :].reshape(N_NODES) + half_b
    s2_ref[...] = r[1:2, :].reshape(N_NODES) + half_b


def _node_projections(h, W, b):
    return pl.pallas_call(
        _node_proj_body,
        in_specs=[
            pl.BlockSpec(memory_space=pltpu.VMEM),
            pl.BlockSpec(memory_space=pltpu.VMEM),
            pl.BlockSpec(memory_space=pltpu.SMEM),
        ],
        out_shape=[
            jax.ShapeDtypeStruct((N_NODES,), jnp.float32),
            jax.ShapeDtypeStruct((N_NODES,), jnp.float32),
        ],
    )(h, W.reshape(2 * D_FEAT), b)


_sc_mesh = plsc.VectorSubcoreMesh(core_axis_name="c", subcore_axis_name="s")


@functools.partial(
    pl.kernel,
    out_type=jax.ShapeDtypeStruct((N_EDGES,), jnp.float32),
    mesh=_sc_mesh,
    compiler_params=pltpu.CompilerParams(needs_layout_passes=False),
    scratch_types=[
        pltpu.VMEM((N_NODES,), jnp.float32),
        pltpu.VMEM((N_NODES,), jnp.float32),
        pltpu.VMEM((2, _BUF_E), jnp.int32),
        pltpu.VMEM((_BUF_E,), jnp.float32),
        pltpu.SemaphoreType.DMA,
    ],
)
def _edge_score_kernel(s1_hbm, s2_hbm, ei_hbm, out_hbm,
                       s1_v, s2_v, ei_v, o_v, sem):
    wid = lax.axis_index("s") * _NC + lax.axis_index("c")
    base = wid * _MAIN_E
    has_tail = wid < 2

    d1 = pltpu.async_copy(s1_hbm, s1_v, sem)
    d2 = pltpu.async_copy(s2_hbm, s2_v, sem)
    d3 = pltpu.async_copy(ei_hbm.at[:, pl.ds(base, _MAIN_E)],
                          ei_v.at[:, pl.ds(0, _MAIN_E)], sem)

    @pl.when(has_tail)
    def _tail_idx():
        pltpu.async_copy(ei_hbm.at[:, pl.ds(_TAIL_E0 + wid * _CT, _CT)],
                         ei_v.at[:, pl.ds(_MAIN_E, _CT)], sem).wait()

    d1.wait()
    d2.wait()
    d3.wait()

    def score_chunk(off):
        si = ei_v[0, pl.ds(off, _L)]
        di = ei_v[1, pl.ds(off, _L)]
        x = plsc.load_gather(s1_v, [si]) + plsc.load_gather(s2_v, [di])
        o_v[pl.ds(off, _L)] = 1.0 / (1.0 + jnp.exp(-x))

    @plsc.parallel_loop(0, _MAIN_CHUNKS, unroll=8)
    def _main(i):
        score_chunk(pl.multiple_of(i * _L, _L))

    @pl.when(has_tail)
    def _tail():
        @plsc.parallel_loop(0, _TAIL_CHUNKS, unroll=8)
        def _t(i):
            score_chunk(pl.multiple_of(_MAIN_E + i * _L, _L))

    pltpu.sync_copy(o_v.at[pl.ds(0, _MAIN_E)],
                    out_hbm.at[pl.ds(base, _MAIN_E)])

    @pl.when(has_tail)
    def _tail_out():
        pltpu.sync_copy(o_v.at[pl.ds(_MAIN_E, _CT)],
                        out_hbm.at[pl.ds(_TAIL_E0 + wid * _CT, _CT)])


def kernel(h, edge_index, W, b):
    s1, s2 = _node_projections(h, W, b)
    return _edge_score_kernel(s1, s2, edge_index)
